# Initial kernel scaffold; baseline (speedup 1.0000x reference)
import jax
import jax.numpy as jnp
from jax.experimental import pallas as pl


def kernel(feat, coord, knn_indexes, params):
    raise NotImplementedError("write your pallas kernel here")


# speedup vs baseline: 2.5622x; 2.5622x over previous
import functools

import jax
import jax.numpy as jnp
from jax import lax
from jax.experimental import pallas as pl
from jax.experimental.pallas import tpu as pltpu
from jax.experimental.pallas import tpu_sc as plsc

N = 10000
K = 32
C = 128
G = 8
L = 16
EPS = 1e-5
E = N * K

_INFO = None


def _sc_info():
    global _INFO
    if _INFO is None:
        _INFO = plsc.get_sparse_core_info()
    return _INFO


def _dense_body(feat_ref, wq_ref, wk_ref, wv_ref, cq_ref, ck_ref, cv_ref,
                w1_ref, b1_ref,
                v_ref, kw_ref, qf_ref):
    f = feat_ref[...]
    colmean = jnp.sum(f, axis=0, keepdims=True) / N
    xtx = lax.dot_general(f, f, (((0,), (0,)), ((), ())),
                          preferred_element_type=jnp.float32) / N

    def proj_bn_relu(W, g, be):
        mu0 = colmean @ W
        var = jnp.sum((xtx @ W) * W, axis=0, keepdims=True) - mu0 * mu0
        sc = g * lax.rsqrt(var + EPS)
        y = lax.dot_general(f, W * sc, (((1,), (0,)), ((), ())),
                            preferred_element_type=jnp.float32)
        return jax.nn.relu(y - mu0 * sc + be)

    cq = cq_ref[...]
    ck = ck_ref[...]
    q = proj_bn_relu(wq_ref[...], cq[1:2], cq[2:3])
    k = proj_bn_relu(wk_ref[...], ck[1:2], ck[2:3])
    cv = cv_ref[...]
    v = lax.dot_general(f, wv_ref[...], (((1,), (0,)), ((), ())),
                        preferred_element_type=jnp.float32) + cv[0:1]
    v_ref[...] = v
    w1 = w1_ref[...]
    kw_ref[...] = lax.dot_general(k, w1, (((1,), (0,)), ((), ())),
                                  preferred_element_type=jnp.float32)
    qf_ref[...] = lax.dot_general(q, w1, (((1,), (0,)), ((), ())),
                                  preferred_element_type=jnp.float32) - b1_ref[...]


def _dense_stage(feat, p):
    cq = jnp.stack([p["b_q"], p["g_q"], p["be_q"], jnp.zeros((C,), jnp.float32)])
    ck = jnp.stack([p["b_k"], p["g_k"], p["be_k"], jnp.zeros((C,), jnp.float32)])
    cv = jnp.stack([p["b_v"], jnp.zeros((C,), jnp.float32),
                    jnp.zeros((C,), jnp.float32), jnp.zeros((C,), jnp.float32)])
    return pl.pallas_call(
        _dense_body,
        out_shape=(
            jax.ShapeDtypeStruct((N, C), jnp.float32),
            jax.ShapeDtypeStruct((N, G), jnp.float32),
            jax.ShapeDtypeStruct((N, G), jnp.float32),
        ),
    )(feat, p["W_q"], p["W_k"], p["W_v"], cq, ck, cv,
      p["W_w1"], p["b_w1"].reshape(1, G))


_KWCH = 2000


def _gather_kw_body(kw_hbm, idx_hbm, out_hbm, kwtab, idx_v, rows_v, sem):
    info = _sc_info()
    nc = info.num_cores
    nw = nc * info.num_subcores
    wid = lax.axis_index("s") * nc + lax.axis_index("c")
    per_w = E // nw
    base = wid * per_w
    pltpu.sync_copy(kw_hbm, kwtab)
    loff = jax.lax.broadcasted_iota(jnp.int32, (L,), 0) % G
    halfsel = jax.lax.broadcasted_iota(jnp.int32, (L,), 0) // G
    pconsts = [halfsel + (2 * pp) for pp in range(G)]

    def chunk(cc, carry):
        e0 = base + cc * _KWCH
        pltpu.sync_copy(idx_hbm.at[pl.ds(e0, _KWCH)], idx_v)

        def u_step(u, carry2):
            idx16 = idx_v[pl.ds(u * L, L)]
            for pp in range(G):
                sel = jnp.take_along_axis(idx16, pconsts[pp], axis=0,
                                          mode="promise_in_bounds")
                gidx = sel * G + loff
                val = plsc.load_gather(kwtab, [gidx])
                rows_v[pl.ds(u * L * G + pp * L, L)] = val
            return carry2

        lax.fori_loop(0, _KWCH // L, u_step, 0)
        pltpu.sync_copy(rows_v, out_hbm.at[pl.ds(e0 * G, _KWCH * G)])
        return carry

    lax.fori_loop(0, per_w // _KWCH, chunk, 0)


def _gather_kw_stage(kw_flat, idx_flat):
    mesh = plsc.VectorSubcoreMesh(core_axis_name="c", subcore_axis_name="s")
    return pl.kernel(
        _gather_kw_body,
        out_type=jax.ShapeDtypeStruct((E * G,), jnp.float32),
        mesh=mesh,
        scratch_types=[
            pltpu.VMEM((N * G,), jnp.float32),
            pltpu.VMEM((_KWCH,), jnp.int32),
            pltpu.VMEM((_KWCH * G,), jnp.float32),
            pltpu.SemaphoreType.DMA,
        ],
        compiler_params=pltpu.CompilerParams(needs_layout_passes=False),
    )(kw_flat, idx_flat)


_BN2 = 200
_NB2 = N // _BN2
_BE2 = _BN2 * K


def _stats_body(gkw_ref, qf_ref, out_ref, acc_ref):
    i = pl.program_id(0)

    @pl.when(i == 0)
    def _():
        acc_ref[...] = jnp.zeros_like(acc_ref)

    a = gkw_ref[...] - qf_ref[...][:, None, :]
    ps = jnp.sum(jnp.sum(a, axis=0), axis=0, keepdims=True)
    ps2 = jnp.sum(jnp.sum(a * a, axis=0), axis=0, keepdims=True)
    acc_ref[0:1, :] += ps
    acc_ref[1:2, :] += ps2

    @pl.when(i == _NB2 - 1)
    def _():
        mean = acc_ref[0:1, :] / E
        var = acc_ref[1:2, :] / E - mean * mean
        inv = lax.rsqrt(var + EPS)
        out_ref[...] = jnp.concatenate(
            [mean, inv, jnp.zeros_like(mean), jnp.zeros_like(mean)] * 2,
            axis=0)


def _stats_stage(gkw3, qf8):
    return pl.pallas_call(
        _stats_body,
        grid=(_NB2,),
        in_specs=[
            pl.BlockSpec((_BN2, K, G), lambda i: (i, 0, 0)),
            pl.BlockSpec((_BN2, G), lambda i: (i, 0)),
        ],
        out_specs=pl.BlockSpec((8, G), lambda i: (0, 0)),
        out_shape=jax.ShapeDtypeStruct((8, G), jnp.float32),
        scratch_shapes=[pltpu.VMEM((8, G), jnp.float32)],
    )(gkw3, qf8)


def _weights_body(gkw_ref, qf_ref, st_ref, wp_ref, out_ref):
    st = st_ref[...]
    wp = wp_ref[...]
    wscale = (wp[0:1, :] * st[1:2, :])[:, None, :]
    wshift = (wp[1:2, :] - wp[0:1, :] * st[1:2, :] * st[0:1, :])[:, None, :]
    a = gkw_ref[...] - qf_ref[...][:, None, :]
    h = jax.nn.relu(a * wscale + wshift)
    logits = jnp.broadcast_to(wp[2:3, :][:, None, :], h.shape)
    for g in range(G):
        logits = logits + h[:, :, g:g + 1] * wp[3 + g:4 + g, :][:, None, :]
    m = jnp.max(logits, axis=1, keepdims=True)
    ex = jnp.exp(logits - m)
    out_ref[...] = ex / jnp.sum(ex, axis=1, keepdims=True)


def _weights_stage(gkw3, qf8, stats, wparams):
    return pl.pallas_call(
        _weights_body,
        grid=(_NB2,),
        in_specs=[
            pl.BlockSpec((_BN2, K, G), lambda i: (i, 0, 0)),
            pl.BlockSpec((_BN2, G), lambda i: (i, 0)),
            pl.BlockSpec((8, G), lambda i: (0, 0)),
            pl.BlockSpec((16, G), lambda i: (0, 0)),
        ],
        out_specs=pl.BlockSpec((_BN2, K, G), lambda i: (i, 0, 0)),
        out_shape=jax.ShapeDtypeStruct((N, K, G), jnp.float32),
    )(gkw3, qf8, stats, wparams)


_AGG_ROWS = 4
_AGG_EDGES = _AGG_ROWS * 128
_AGG_NODES = _AGG_EDGES // K
_AGG_NCHUNK = N // _AGG_NODES


def _agg_body(v_hbm, idx_hbm, w_hbm, out_hbm, idx_v, vbuf, wbuf, obuf, sem):
    info = _sc_info()
    nc = info.num_cores
    nw = nc * info.num_subcores
    wid = lax.axis_index("s") * nc + lax.axis_index("c")
    trips = (_AGG_NCHUNK + nw - 1) // nw
    bconsts = [jnp.full((L,), j, jnp.int32) for j in range(2 * G)]

    def trip(t, carry):
        c = wid + t * nw

        @pl.when(c < _AGG_NCHUNK)
        def _():
            e0 = c * _AGG_EDGES
            pltpu.sync_copy(idx_hbm.at[pl.ds(e0, _AGG_EDGES)], idx_v)
            descs = [
                pltpu.async_copy(v_hbm.at[idx_v.at[pl.ds(j * 128, 128)]],
                                 vbuf.at[pl.ds(j * 128, 128)], sem)
                for j in range(_AGG_ROWS)
            ]
            pltpu.sync_copy(w_hbm.at[pl.ds(e0 * G, _AGG_EDGES * G)], wbuf)
            for d in descs:
                d.wait()

            def node(ni, carry2):
                accs = [jnp.zeros((L,), jnp.float32) for _ in range(G)]
                for t2 in range(K // 2):
                    r = ni * K + 2 * t2
                    wv = wbuf[pl.ds(r * G, L)]
                    for g in range(G):
                        b0 = jnp.take_along_axis(wv, bconsts[g], axis=0,
                                                 mode="promise_in_bounds")
                        b1 = jnp.take_along_axis(wv, bconsts[G + g], axis=0,
                                                 mode="promise_in_bounds")
                        accs[g] = (accs[g]
                                   + b0 * vbuf[r, pl.ds(g * L, L)]
                                   + b1 * vbuf[r + 1, pl.ds(g * L, L)])
                for g in range(G):
                    obuf[pl.ds(ni * C + g * L, L)] = accs[g]
                return carry2

            lax.fori_loop(0, _AGG_NODES, node, 0)
            pltpu.sync_copy(obuf, out_hbm.at[pl.ds(c * _AGG_NODES * C,
                                                   _AGG_NODES * C)])
        return carry

    lax.fori_loop(0, trips, trip, 0)


def _agg_stage(v, idx_flat, wflat):
    mesh = plsc.VectorSubcoreMesh(core_axis_name="c", subcore_axis_name="s")
    return pl.kernel(
        _agg_body,
        out_type=jax.ShapeDtypeStruct((N * C,), jnp.float32),
        mesh=mesh,
        scratch_types=[
            pltpu.VMEM((_AGG_EDGES,), jnp.int32),
            pltpu.VMEM((_AGG_EDGES, C), jnp.float32),
            pltpu.VMEM((_AGG_EDGES * G,), jnp.float32),
            pltpu.VMEM((_AGG_NODES * C,), jnp.float32),
            pltpu.SemaphoreType.DMA,
        ],
    )(v, idx_flat, wflat)


def kernel(feat, coord, knn_indexes, params):
    del coord
    p = params
    idx_flat = jnp.reshape(knn_indexes.astype(jnp.int32), (E,))

    v, kw8, qf8 = _dense_stage(feat, p)
    gkw = _gather_kw_stage(jnp.reshape(kw8, (-1,)), idx_flat)
    gkw3 = jnp.reshape(gkw, (N, K, G))

    wparams = jnp.concatenate([
        p["g_w"].reshape(1, G), p["be_w"].reshape(1, G),
        p["b_w2"].reshape(1, G), p["W_w2"],
    ], axis=0)
    wparams = jnp.pad(wparams, ((0, 5), (0, 0)))

    stats = _stats_stage(gkw3, qf8)
    w = _weights_stage(gkw3, qf8, stats, wparams)

    out = _agg_stage(v, idx_flat, jnp.reshape(w, (-1,)))
    return jnp.reshape(out, (N, C))


# speedup vs baseline: 5.8652x; 2.2891x over previous
import functools

import jax
import jax.numpy as jnp
from jax import lax
from jax.experimental import pallas as pl
from jax.experimental.pallas import tpu as pltpu
from jax.experimental.pallas import tpu_sc as plsc

N = 10000
K = 32
C = 128
G = 8
L = 16
EPS = 1e-5
E = N * K

_INFO = None


def _sc_info():
    global _INFO
    if _INFO is None:
        _INFO = plsc.get_sparse_core_info()
    return _INFO


def _dense_body(feat_ref, wq_ref, wk_ref, wv_ref, cq_ref, ck_ref, cv_ref,
                w1_ref, w1t_ref, b1_ref,
                v_ref, kw_ref, qf_ref):
    f = feat_ref[...]
    colmean = jnp.sum(f, axis=0, keepdims=True) / N
    xtx = lax.dot_general(f, f, (((0,), (0,)), ((), ())),
                          preferred_element_type=jnp.float32) / N

    def proj_bn_relu(W, g, be):
        mu0 = colmean @ W
        var = jnp.sum((xtx @ W) * W, axis=0, keepdims=True) - mu0 * mu0
        sc = g * lax.rsqrt(var + EPS)
        y = lax.dot_general(f, W * sc, (((1,), (0,)), ((), ())),
                            preferred_element_type=jnp.float32)
        return jax.nn.relu(y - mu0 * sc + be)

    cq = cq_ref[...]
    ck = ck_ref[...]
    q = proj_bn_relu(wq_ref[...], cq[1:2], cq[2:3])
    k = proj_bn_relu(wk_ref[...], ck[1:2], ck[2:3])
    cv = cv_ref[...]
    v = lax.dot_general(f, wv_ref[...], (((1,), (0,)), ((), ())),
                        preferred_element_type=jnp.float32) + cv[0:1]
    v_ref[...] = v
    w1 = w1_ref[...]
    kw_ref[...] = lax.dot_general(k, w1, (((1,), (0,)), ((), ())),
                                  preferred_element_type=jnp.float32)
    qf_ref[...] = lax.dot_general(q, w1t_ref[...], (((1,), (0,)), ((), ())),
                                  preferred_element_type=jnp.float32) - b1_ref[...]


def _dense_stage(feat, p):
    cq = jnp.stack([p["b_q"], p["g_q"], p["be_q"], jnp.zeros((C,), jnp.float32)])
    ck = jnp.stack([p["b_k"], p["g_k"], p["be_k"], jnp.zeros((C,), jnp.float32)])
    cv = jnp.stack([p["b_v"], jnp.zeros((C,), jnp.float32),
                    jnp.zeros((C,), jnp.float32), jnp.zeros((C,), jnp.float32)])
    return pl.pallas_call(
        _dense_body,
        out_shape=(
            jax.ShapeDtypeStruct((N, C), jnp.float32),
            jax.ShapeDtypeStruct((N, G), jnp.float32),
            jax.ShapeDtypeStruct((N, C), jnp.float32),
        ),
    )(feat, p["W_q"], p["W_k"], p["W_v"], cq, ck, cv,
      p["W_w1"], jnp.tile(p["W_w1"], (1, L)),
      jnp.tile(p["b_w1"], L).reshape(1, C))


_KWCH = 2000


def _gather_kw_body(kw_hbm, idx_hbm, out_hbm, kwtab, idx_v, rows_v, sem):
    info = _sc_info()
    nc = info.num_cores
    nw = nc * info.num_subcores
    wid = lax.axis_index("s") * nc + lax.axis_index("c")
    per_w = E // nw
    base = wid * per_w
    pltpu.sync_copy(kw_hbm, kwtab)
    loff = jax.lax.broadcasted_iota(jnp.int32, (L,), 0) % G
    halfsel = jax.lax.broadcasted_iota(jnp.int32, (L,), 0) // G
    pconsts = [halfsel + (2 * pp) for pp in range(G)]

    def chunk(cc, carry):
        e0 = base + cc * _KWCH
        pltpu.sync_copy(idx_hbm.at[pl.ds(e0, _KWCH)], idx_v)

        def u_step(u, carry2):
            idx16 = idx_v[pl.ds(u * L, L)]
            for pp in range(G):
                sel = jnp.take_along_axis(idx16, pconsts[pp], axis=0,
                                          mode="promise_in_bounds")
                gidx = sel * G + loff
                val = plsc.load_gather(kwtab, [gidx])
                rows_v[pl.ds(u * L * G + pp * L, L)] = val
            return carry2

        lax.fori_loop(0, _KWCH // L, u_step, 0)
        pltpu.sync_copy(rows_v, out_hbm.at[pl.ds(e0 * G, _KWCH * G)])
        return carry

    lax.fori_loop(0, per_w // _KWCH, chunk, 0)


def _gather_kw_stage(kw_flat, idx_flat):
    mesh = plsc.VectorSubcoreMesh(core_axis_name="c", subcore_axis_name="s")
    return pl.kernel(
        _gather_kw_body,
        out_type=jax.ShapeDtypeStruct((E * G,), jnp.float32),
        mesh=mesh,
        scratch_types=[
            pltpu.VMEM((N * G,), jnp.float32),
            pltpu.VMEM((_KWCH,), jnp.int32),
            pltpu.VMEM((_KWCH * G,), jnp.float32),
            pltpu.SemaphoreType.DMA,
        ],
        compiler_params=pltpu.CompilerParams(needs_layout_passes=False),
    )(kw_flat, idx_flat)


_BN2 = 1000
_NB2 = N // _BN2


def _lane_rot(x, sh):
    shp = x.shape
    n = shp[-1]
    sq = tuple(d for d in shp[:-1] if d != 1) + (n,)
    xs = jnp.reshape(x, sq)
    iota = lax.broadcasted_iota(jnp.int32, (1,) * (len(sq) - 1) + (n,),
                                len(sq) - 1)
    idx = (iota + sh) % n
    r = jnp.take_along_axis(xs, jnp.broadcast_to(idx, sq),
                            axis=len(sq) - 1, mode="promise_in_bounds")
    return jnp.reshape(r, shp)


def _fold16(x, op):
    for sh in (8, 16, 32, 64):
        x = op(x, _lane_rot(x, sh))
    return x


def _stats_body(gkw_ref, qf_ref, gbe_ref, out_ref, acc_ref):
    i = pl.program_id(0)

    @pl.when(i == 0)
    def _():
        acc_ref[...] = jnp.zeros_like(acc_ref)

    a = gkw_ref[...] - qf_ref[...][:, None, :]
    ps = jnp.sum(jnp.sum(a, axis=0), axis=0, keepdims=True)
    ps2 = jnp.sum(jnp.sum(a * a, axis=0), axis=0, keepdims=True)
    acc_ref[0:1, :] += ps
    acc_ref[1:2, :] += ps2

    @pl.when(i == _NB2 - 1)
    def _():
        accf = _fold16(acc_ref[...], jnp.add)
        mean = accf[0:1, :] / E
        var = accf[1:2, :] / E - mean * mean
        ws = gbe_ref[0:1, :] * lax.rsqrt(var + EPS)
        wsh = gbe_ref[1:2, :] - ws * mean
        out_ref[...] = jnp.concatenate(
            [ws, wsh, jnp.zeros_like(ws), jnp.zeros_like(ws)] * 2, axis=0)


def _stats_stage(gkw2, qft, gbe):
    return pl.pallas_call(
        _stats_body,
        grid=(_NB2,),
        in_specs=[
            pl.BlockSpec((_BN2, 2, C), lambda i: (i, 0, 0)),
            pl.BlockSpec((_BN2, C), lambda i: (i, 0)),
            pl.BlockSpec((2, C), lambda i: (0, 0)),
        ],
        out_specs=pl.BlockSpec((8, C), lambda i: (0, 0)),
        out_shape=jax.ShapeDtypeStruct((8, C), jnp.float32),
        scratch_shapes=[pltpu.VMEM((8, C), jnp.float32)],
    )(gkw2, qft, gbe)


def _weights_body(gkw_ref, qf_ref, st_ref, w2_ref, out_ref):
    wscale = st_ref[0:1, :][:, None, :]
    wshift = st_ref[1:2, :][:, None, :]
    a = gkw_ref[...] - qf_ref[...][:, None, :]
    h = jax.nn.relu(a * wscale + wshift)
    base = (lax.broadcasted_iota(jnp.int32, (1, 1, C), 2) // G) * G
    logits = jnp.broadcast_to(w2_ref[G:G + 1, :][:, None, :], h.shape)
    for g in range(G):
        blg = jnp.take_along_axis(h, jnp.broadcast_to(base + g, h.shape),
                                  axis=2, mode="promise_in_bounds")
        logits = logits + blg * w2_ref[g:g + 1, :][:, None, :]
    m = _fold16(jnp.max(logits, axis=1, keepdims=True), jnp.maximum)
    ex = jnp.exp(logits - m)
    s = _fold16(jnp.sum(ex, axis=1, keepdims=True), jnp.add)
    out_ref[...] = ex / s


def _weights_stage(gkw2, qft, stats, w2b):
    return pl.pallas_call(
        _weights_body,
        grid=(_NB2,),
        in_specs=[
            pl.BlockSpec((_BN2, 2, C), lambda i: (i, 0, 0)),
            pl.BlockSpec((_BN2, C), lambda i: (i, 0)),
            pl.BlockSpec((8, C), lambda i: (0, 0)),
            pl.BlockSpec((16, C), lambda i: (0, 0)),
        ],
        out_specs=pl.BlockSpec((_BN2, 2, C), lambda i: (i, 0, 0)),
        out_shape=jax.ShapeDtypeStruct((N, 2, C), jnp.float32),
    )(gkw2, qft, stats, w2b)


_AGG_ROWS = 4
_AGG_EDGES = _AGG_ROWS * 128
_AGG_NODES = _AGG_EDGES // K
_AGG_NCHUNK = N // _AGG_NODES


def _agg_body(v_hbm, idx_hbm, w_hbm, out_hbm, idx_v, vbuf, wbuf, obuf, sem):
    info = _sc_info()
    nc = info.num_cores
    nw = nc * info.num_subcores
    wid = lax.axis_index("s") * nc + lax.axis_index("c")
    trips = (_AGG_NCHUNK + nw - 1) // nw
    bconsts = [jnp.full((L,), j, jnp.int32) for j in range(2 * G)]

    def trip(t, carry):
        c = wid + t * nw

        @pl.when(c < _AGG_NCHUNK)
        def _():
            e0 = c * _AGG_EDGES
            pltpu.sync_copy(idx_hbm.at[pl.ds(e0, _AGG_EDGES)], idx_v)
            descs = [
                pltpu.async_copy(v_hbm.at[idx_v.at[pl.ds(j * 128, 128)]],
                                 vbuf.at[pl.ds(j * 128, 128)], sem)
                for j in range(_AGG_ROWS)
            ]
            pltpu.sync_copy(w_hbm.at[pl.ds(e0 * G, _AGG_EDGES * G)], wbuf)
            for d in descs:
                d.wait()

            def node(ni, carry2):
                accs = [jnp.zeros((L,), jnp.float32) for _ in range(G)]
                for t2 in range(K // 2):
                    r = ni * K + 2 * t2
                    wv = wbuf[pl.ds(r * G, L)]
                    for g in range(G):
                        b0 = jnp.take_along_axis(wv, bconsts[g], axis=0,
                                                 mode="promise_in_bounds")
                        b1 = jnp.take_along_axis(wv, bconsts[G + g], axis=0,
                                                 mode="promise_in_bounds")
                        accs[g] = (accs[g]
                                   + b0 * vbuf[r, pl.ds(g * L, L)]
                                   + b1 * vbuf[r + 1, pl.ds(g * L, L)])
                for g in range(G):
                    obuf[pl.ds(ni * C + g * L, L)] = accs[g]
                return carry2

            lax.fori_loop(0, _AGG_NODES, node, 0)
            pltpu.sync_copy(obuf, out_hbm.at[pl.ds(c * _AGG_NODES * C,
                                                   _AGG_NODES * C)])
        return carry

    lax.fori_loop(0, trips, trip, 0)


def _agg_stage(v, idx_flat, wflat):
    mesh = plsc.VectorSubcoreMesh(core_axis_name="c", subcore_axis_name="s")
    return pl.kernel(
        _agg_body,
        out_type=jax.ShapeDtypeStruct((N * C,), jnp.float32),
        mesh=mesh,
        scratch_types=[
            pltpu.VMEM((_AGG_EDGES,), jnp.int32),
            pltpu.VMEM((_AGG_EDGES, C), jnp.float32),
            pltpu.VMEM((_AGG_EDGES * G,), jnp.float32),
            pltpu.VMEM((_AGG_NODES * C,), jnp.float32),
            pltpu.SemaphoreType.DMA,
        ],
    )(v, idx_flat, wflat)


def kernel(feat, coord, knn_indexes, params):
    del coord
    p = params
    idx_flat = jnp.reshape(knn_indexes.astype(jnp.int32), (E,))

    v, kw8, qft = _dense_stage(feat, p)
    gkw = _gather_kw_stage(jnp.reshape(kw8, (-1,)), idx_flat)
    gkw2 = jnp.reshape(gkw, (N, 2, C))

    gbe = jnp.stack([jnp.tile(p["g_w"], L), jnp.tile(p["be_w"], L)])
    w2b = jnp.concatenate([
        jnp.tile(p["W_w2"], (1, L)),
        jnp.tile(p["b_w2"], L).reshape(1, C),
        jnp.zeros((7, C), jnp.float32),
    ], axis=0)

    stats = _stats_stage(gkw2, qft, gbe)
    w = _weights_stage(gkw2, qft, stats, w2b)

    out = _agg_stage(v, idx_flat, jnp.reshape(w, (-1,)))
    return jnp.reshape(out, (N, C))


# speedup vs baseline: 9.4672x; 1.6141x over previous
import functools

import jax
import jax.numpy as jnp
from jax import lax
from jax.experimental import pallas as pl
from jax.experimental.pallas import tpu as pltpu
from jax.experimental.pallas import tpu_sc as plsc

N = 10000
K = 32
C = 128
G = 8
L = 16
EPS = 1e-5
E = N * K

_INFO = None


def _sc_info():
    global _INFO
    if _INFO is None:
        _INFO = plsc.get_sparse_core_info()
    return _INFO


def _dense_body(feat_ref, wq_ref, wk_ref, wv_ref, cq_ref, ck_ref, cv_ref,
                w1_ref, w1t_ref, b1_ref,
                v_ref, kw_ref, qf_ref):
    f = feat_ref[...]
    colmean = jnp.sum(f, axis=0, keepdims=True) / N
    xtx = lax.dot_general(f, f, (((0,), (0,)), ((), ())),
                          preferred_element_type=jnp.float32) / N

    def proj_bn_relu(W, g, be):
        mu0 = colmean @ W
        var = jnp.sum((xtx @ W) * W, axis=0, keepdims=True) - mu0 * mu0
        sc = g * lax.rsqrt(var + EPS)
        y = lax.dot_general(f, W * sc, (((1,), (0,)), ((), ())),
                            preferred_element_type=jnp.float32)
        return jax.nn.relu(y - mu0 * sc + be)

    cq = cq_ref[...]
    ck = ck_ref[...]
    q = proj_bn_relu(wq_ref[...], cq[1:2], cq[2:3])
    k = proj_bn_relu(wk_ref[...], ck[1:2], ck[2:3])
    cv = cv_ref[...]
    v = lax.dot_general(f, wv_ref[...], (((1,), (0,)), ((), ())),
                        preferred_element_type=jnp.float32) + cv[0:1]
    v_ref[...] = v
    w1 = w1_ref[...]
    kw_ref[...] = lax.dot_general(k, w1, (((1,), (0,)), ((), ())),
                                  preferred_element_type=jnp.float32)
    qf_ref[...] = lax.dot_general(q, w1t_ref[...], (((1,), (0,)), ((), ())),
                                  preferred_element_type=jnp.float32) - b1_ref[...]


def _dense_stage(feat, p):
    cq = jnp.stack([p["b_q"], p["g_q"], p["be_q"], jnp.zeros((C,), jnp.float32)])
    ck = jnp.stack([p["b_k"], p["g_k"], p["be_k"], jnp.zeros((C,), jnp.float32)])
    cv = jnp.stack([p["b_v"], jnp.zeros((C,), jnp.float32),
                    jnp.zeros((C,), jnp.float32), jnp.zeros((C,), jnp.float32)])
    return pl.pallas_call(
        _dense_body,
        out_shape=(
            jax.ShapeDtypeStruct((N, C), jnp.float32),
            jax.ShapeDtypeStruct((N, G), jnp.float32),
            jax.ShapeDtypeStruct((N, C), jnp.float32),
        ),
    )(feat, p["W_q"], p["W_k"], p["W_v"], cq, ck, cv,
      p["W_w1"], jnp.tile(p["W_w1"], (1, L)),
      jnp.tile(p["b_w1"], L).reshape(1, C))


_KWCH = 2000


def _gather_kw_body(kw_hbm, idx_hbm, out_hbm, kwtab, idx_v, rows_v, sem):
    info = _sc_info()
    nc = info.num_cores
    nw = nc * info.num_subcores
    wid = lax.axis_index("s") * nc + lax.axis_index("c")
    per_w = E // nw
    base = wid * per_w
    pltpu.sync_copy(kw_hbm, kwtab)
    loff = jax.lax.broadcasted_iota(jnp.int32, (L,), 0) % G
    halfsel = jax.lax.broadcasted_iota(jnp.int32, (L,), 0) // G
    pconsts = [halfsel + (2 * pp) for pp in range(G)]

    def chunk(cc, carry):
        e0 = base + cc * _KWCH
        pltpu.sync_copy(idx_hbm.at[pl.ds(e0, _KWCH)], idx_v)

        def u_step(u, carry2):
            idx16 = idx_v[pl.ds(u * L, L)]
            for pp in range(G):
                sel = jnp.take_along_axis(idx16, pconsts[pp], axis=0,
                                          mode="promise_in_bounds")
                gidx = sel * G + loff
                val = plsc.load_gather(kwtab, [gidx])
                rows_v[pl.ds(u * L * G + pp * L, L)] = val
            return carry2

        lax.fori_loop(0, _KWCH // L, u_step, 0)
        pltpu.sync_copy(rows_v, out_hbm.at[pl.ds(e0 * G, _KWCH * G)])
        return carry

    lax.fori_loop(0, per_w // _KWCH, chunk, 0)


def _gather_kw_stage(kw_flat, idx_flat):
    mesh = plsc.VectorSubcoreMesh(core_axis_name="c", subcore_axis_name="s")
    return pl.kernel(
        _gather_kw_body,
        out_type=jax.ShapeDtypeStruct((E * G,), jnp.float32),
        mesh=mesh,
        scratch_types=[
            pltpu.VMEM((N * G,), jnp.float32),
            pltpu.VMEM((_KWCH,), jnp.int32),
            pltpu.VMEM((_KWCH * G,), jnp.float32),
            pltpu.SemaphoreType.DMA,
        ],
        compiler_params=pltpu.CompilerParams(needs_layout_passes=False),
    )(kw_flat, idx_flat)


_BN2 = 1000
_NB2 = N // _BN2


def _lane_rot(x, sh):
    shp = x.shape
    n = shp[-1]
    sq = tuple(d for d in shp[:-1] if d != 1) + (n,)
    xs = jnp.reshape(x, sq)
    iota = lax.broadcasted_iota(jnp.int32, (1,) * (len(sq) - 1) + (n,),
                                len(sq) - 1)
    idx = (iota + sh) % n
    r = jnp.take_along_axis(xs, jnp.broadcast_to(idx, sq),
                            axis=len(sq) - 1, mode="promise_in_bounds")
    return jnp.reshape(r, shp)


def _fold16(x, op):
    for sh in (8, 16, 32, 64):
        x = op(x, _lane_rot(x, sh))
    return x


def _stats_body(gkw_ref, qf_ref, gbe_ref, out_ref, acc_ref):
    i = pl.program_id(0)

    @pl.when(i == 0)
    def _():
        acc_ref[...] = jnp.zeros_like(acc_ref)

    a = gkw_ref[...] - qf_ref[...][:, None, :]
    ps = jnp.sum(jnp.sum(a, axis=0), axis=0, keepdims=True)
    ps2 = jnp.sum(jnp.sum(a * a, axis=0), axis=0, keepdims=True)
    acc_ref[0:1, :] += ps
    acc_ref[1:2, :] += ps2

    @pl.when(i == _NB2 - 1)
    def _():
        accf = _fold16(acc_ref[...], jnp.add)
        mean = accf[0:1, :] / E
        var = accf[1:2, :] / E - mean * mean
        ws = gbe_ref[0:1, :] * lax.rsqrt(var + EPS)
        wsh = gbe_ref[1:2, :] - ws * mean
        out_ref[...] = jnp.concatenate(
            [ws, wsh, jnp.zeros_like(ws), jnp.zeros_like(ws)] * 2, axis=0)


def _stats_stage(gkw2, qft, gbe):
    return pl.pallas_call(
        _stats_body,
        grid=(_NB2,),
        in_specs=[
            pl.BlockSpec((_BN2, 2, C), lambda i: (i, 0, 0)),
            pl.BlockSpec((_BN2, C), lambda i: (i, 0)),
            pl.BlockSpec((2, C), lambda i: (0, 0)),
        ],
        out_specs=pl.BlockSpec((8, C), lambda i: (0, 0)),
        out_shape=jax.ShapeDtypeStruct((8, C), jnp.float32),
        scratch_shapes=[pltpu.VMEM((8, C), jnp.float32)],
    )(gkw2, qft, gbe)


def _weights_body(gkw_ref, qf_ref, st_ref, m2_ref, sm_ref, b2_ref, out_ref):
    wscale = st_ref[0:1, :][:, None, :]
    wshift = st_ref[1:2, :][:, None, :]
    a = gkw_ref[...] - qf_ref[...][:, None, :]
    h = jax.nn.relu(a * wscale + wshift)
    h2 = jnp.reshape(h, (_BN2 * 2, C))
    logits2 = lax.dot_general(h2, m2_ref[...], (((1,), (0,)), ((), ())),
                              preferred_element_type=jnp.float32) + b2_ref[...]
    logits = jnp.reshape(logits2, (_BN2, 2, C))
    mx = _fold16(jnp.max(logits, axis=1), jnp.maximum)
    ex = jnp.exp(logits - mx[:, None, :])
    es = ex[:, 0, :] + ex[:, 1, :]
    s = lax.dot_general(es, sm_ref[...], (((1,), (0,)), ((), ())),
                        preferred_element_type=jnp.float32)
    out_ref[...] = ex / s[:, None, :]


def _weights_stage(gkw2, qft, stats, m2, sm, b2t):
    return pl.pallas_call(
        _weights_body,
        grid=(_NB2,),
        in_specs=[
            pl.BlockSpec((_BN2, 2, C), lambda i: (i, 0, 0)),
            pl.BlockSpec((_BN2, C), lambda i: (i, 0)),
            pl.BlockSpec((8, C), lambda i: (0, 0)),
            pl.BlockSpec((C, C), lambda i: (0, 0)),
            pl.BlockSpec((C, C), lambda i: (0, 0)),
            pl.BlockSpec((1, C), lambda i: (0, 0)),
        ],
        out_specs=pl.BlockSpec((_BN2, 2, C), lambda i: (i, 0, 0)),
        out_shape=jax.ShapeDtypeStruct((N, 2, C), jnp.float32),
    )(gkw2, qft, stats, m2, sm, b2t)


_AGG_ROWS = 2
_AGG_EDGES = _AGG_ROWS * 128
_AGG_NODES = _AGG_EDGES // K
_AGG_NCHUNK = N // _AGG_NODES


def _agg_body(v_hbm, idx_hbm, w_hbm, out_hbm,
              idx_a, idx_b, vbuf_a, vbuf_b, wbuf_a, wbuf_b, obuf,
              sem_a, sem_b):
    info = _sc_info()
    nc = info.num_cores
    nw = nc * info.num_subcores
    wid = lax.axis_index("s") * nc + lax.axis_index("c")
    bconsts = [jnp.full((L,), j, jnp.int32) for j in range(2 * G)]

    def prefetch(c, idx_v, vbuf, wbuf, sem):
        e0 = c * _AGG_EDGES
        pltpu.sync_copy(idx_hbm.at[pl.ds(e0, _AGG_EDGES)], idx_v)
        for j in range(_AGG_ROWS):
            pltpu.async_copy(v_hbm.at[idx_v.at[pl.ds(j * 128, 128)]],
                             vbuf.at[pl.ds(j * 128, 128)], sem)
        pltpu.async_copy(w_hbm.at[pl.ds(e0 * G, _AGG_EDGES * G)], wbuf, sem)

    def wait_prefetch(idx_v, vbuf, wbuf, sem):
        for j in range(_AGG_ROWS):
            pltpu.make_async_copy(v_hbm.at[idx_v.at[pl.ds(j * 128, 128)]],
                                  vbuf.at[pl.ds(j * 128, 128)], sem).wait()
        pltpu.make_async_copy(w_hbm.at[pl.ds(0, _AGG_EDGES * G)], wbuf,
                              sem).wait()

    def compute(c, vbuf, wbuf):
        def node(ni, carry2):
            accs = [jnp.zeros((L,), jnp.float32) for _ in range(G)]
            for t2 in range(K // 2):
                r = ni * K + 2 * t2
                wv = wbuf[pl.ds(r * G, L)]
                for g in range(G):
                    b0 = jnp.take_along_axis(wv, bconsts[g], axis=0,
                                             mode="promise_in_bounds")
                    b1 = jnp.take_along_axis(wv, bconsts[G + g], axis=0,
                                             mode="promise_in_bounds")
                    accs[g] = (accs[g]
                               + b0 * vbuf[r, pl.ds(g * L, L)]
                               + b1 * vbuf[r + 1, pl.ds(g * L, L)])
            for g in range(G):
                obuf[pl.ds(ni * C + g * L, L)] = accs[g]
            return carry2

        lax.fori_loop(0, _AGG_NODES, node, 0)
        pltpu.sync_copy(obuf, out_hbm.at[pl.ds(c * _AGG_NODES * C,
                                               _AGG_NODES * C)])

    @pl.when(wid < _AGG_NCHUNK)
    def _():
        prefetch(wid, idx_a, vbuf_a, wbuf_a, sem_a)

    pairs = (_AGG_NCHUNK + 2 * nw - 1) // (2 * nw)

    def pair(tt, carry):
        c_a = wid + (2 * tt) * nw
        c_b = c_a + nw
        c_n = c_a + 2 * nw

        @pl.when(c_b < _AGG_NCHUNK)
        def _():
            prefetch(c_b, idx_b, vbuf_b, wbuf_b, sem_b)

        @pl.when(c_a < _AGG_NCHUNK)
        def _():
            wait_prefetch(idx_a, vbuf_a, wbuf_a, sem_a)
            compute(c_a, vbuf_a, wbuf_a)

        @pl.when(c_n < _AGG_NCHUNK)
        def _():
            prefetch(c_n, idx_a, vbuf_a, wbuf_a, sem_a)

        @pl.when(c_b < _AGG_NCHUNK)
        def _():
            wait_prefetch(idx_b, vbuf_b, wbuf_b, sem_b)
            compute(c_b, vbuf_b, wbuf_b)

        return carry

    lax.fori_loop(0, pairs, pair, 0)


def _agg_stage(v, idx_flat, wflat):
    mesh = plsc.VectorSubcoreMesh(core_axis_name="c", subcore_axis_name="s")
    return pl.kernel(
        _agg_body,
        out_type=jax.ShapeDtypeStruct((N * C,), jnp.float32),
        mesh=mesh,
        scratch_types=[
            pltpu.VMEM((_AGG_EDGES,), jnp.int32),
            pltpu.VMEM((_AGG_EDGES,), jnp.int32),
            pltpu.VMEM((_AGG_EDGES, C), jnp.float32),
            pltpu.VMEM((_AGG_EDGES, C), jnp.float32),
            pltpu.VMEM((_AGG_EDGES * G,), jnp.float32),
            pltpu.VMEM((_AGG_EDGES * G,), jnp.float32),
            pltpu.VMEM((_AGG_NODES * C,), jnp.float32),
            pltpu.SemaphoreType.DMA,
            pltpu.SemaphoreType.DMA,
        ],
    )(v, idx_flat, wflat)


def kernel(feat, coord, knn_indexes, params):
    del coord
    p = params
    idx_flat = jnp.reshape(knn_indexes.astype(jnp.int32), (E,))

    v, kw8, qft = _dense_stage(feat, p)
    gkw = _gather_kw_stage(jnp.reshape(kw8, (-1,)), idx_flat)
    gkw2 = jnp.reshape(gkw, (N, 2, C))

    gbe = jnp.stack([jnp.tile(p["g_w"], L), jnp.tile(p["be_w"], L)])
    m2 = jnp.kron(jnp.eye(L, dtype=jnp.float32), p["W_w2"])
    sm = jnp.kron(jnp.ones((L, L), jnp.float32), jnp.eye(G, dtype=jnp.float32))
    b2t = jnp.tile(p["b_w2"], L).reshape(1, C)

    stats = _stats_stage(gkw2, qft, gbe)
    w = _weights_stage(gkw2, qft, stats, m2, sm, b2t)

    out = _agg_stage(v, idx_flat, jnp.reshape(w, (-1,)))
    return jnp.reshape(out, (N, C))


# speedup vs baseline: 10.8482x; 1.1459x over previous
import functools

import jax
import jax.numpy as jnp
from jax import lax
from jax.experimental import pallas as pl
from jax.experimental.pallas import tpu as pltpu
from jax.experimental.pallas import tpu_sc as plsc

N = 10000
K = 32
C = 128
G = 8
L = 16
EPS = 1e-5
E = N * K

_INFO = None


def _sc_info():
    global _INFO
    if _INFO is None:
        _INFO = plsc.get_sparse_core_info()
    return _INFO


def _dense_body(feat_ref, wq_ref, wk_ref, wv_ref, cq_ref, ck_ref, cv_ref,
                w1_ref, w1t_ref, b1_ref,
                v_ref, kw_ref, qf_ref):
    f = feat_ref[...]
    colmean = jnp.sum(f, axis=0, keepdims=True) / N
    xtx = lax.dot_general(f, f, (((0,), (0,)), ((), ())),
                          preferred_element_type=jnp.float32) / N

    def proj_bn_relu(W, g, be):
        mu0 = colmean @ W
        var = jnp.sum((xtx @ W) * W, axis=0, keepdims=True) - mu0 * mu0
        sc = g * lax.rsqrt(var + EPS)
        y = lax.dot_general(f, W * sc, (((1,), (0,)), ((), ())),
                            preferred_element_type=jnp.float32)
        return jax.nn.relu(y - mu0 * sc + be)

    cq = cq_ref[...]
    ck = ck_ref[...]
    q = proj_bn_relu(wq_ref[...], cq[1:2], cq[2:3])
    k = proj_bn_relu(wk_ref[...], ck[1:2], ck[2:3])
    cv = cv_ref[...]
    v = lax.dot_general(f, wv_ref[...], (((1,), (0,)), ((), ())),
                        preferred_element_type=jnp.float32) + cv[0:1]
    v_ref[...] = v
    w1 = w1_ref[...]
    kw_ref[...] = lax.dot_general(k, w1, (((1,), (0,)), ((), ())),
                                  preferred_element_type=jnp.float32)
    qf_ref[...] = lax.dot_general(q, w1t_ref[...], (((1,), (0,)), ((), ())),
                                  preferred_element_type=jnp.float32) - b1_ref[...]


def _dense_stage(feat, p):
    cq = jnp.stack([p["b_q"], p["g_q"], p["be_q"], jnp.zeros((C,), jnp.float32)])
    ck = jnp.stack([p["b_k"], p["g_k"], p["be_k"], jnp.zeros((C,), jnp.float32)])
    cv = jnp.stack([p["b_v"], jnp.zeros((C,), jnp.float32),
                    jnp.zeros((C,), jnp.float32), jnp.zeros((C,), jnp.float32)])
    return pl.pallas_call(
        _dense_body,
        out_shape=(
            jax.ShapeDtypeStruct((N, C), jnp.float32),
            jax.ShapeDtypeStruct((N, G), jnp.float32),
            jax.ShapeDtypeStruct((N, C), jnp.float32),
        ),
    )(feat, p["W_q"], p["W_k"], p["W_v"], cq, ck, cv,
      p["W_w1"], jnp.tile(p["W_w1"], (1, L)),
      jnp.tile(p["b_w1"], L).reshape(1, C))


_KWCH = 2000


def _gather_kw_body(kw_hbm, idx_hbm, out_hbm, kwtab, idx_v, rows_v, sem):
    info = _sc_info()
    nc = info.num_cores
    nw = nc * info.num_subcores
    wid = lax.axis_index("s") * nc + lax.axis_index("c")
    per_w = E // nw
    base = wid * per_w
    pltpu.sync_copy(kw_hbm, kwtab)
    loff = jax.lax.broadcasted_iota(jnp.int32, (L,), 0) % G
    halfsel = jax.lax.broadcasted_iota(jnp.int32, (L,), 0) // G
    pconsts = [halfsel + (2 * pp) for pp in range(G)]

    def chunk(cc, carry):
        e0 = base + cc * _KWCH
        pltpu.sync_copy(idx_hbm.at[pl.ds(e0, _KWCH)], idx_v)

        def u_step(u4, carry2):
            for uu in range(4):
                u = u4 * 4 + uu
                idx16 = idx_v[pl.ds(u * L, L)]
                for pp in range(G):
                    sel = jnp.take_along_axis(idx16, pconsts[pp], axis=0,
                                              mode="promise_in_bounds")
                    gidx = sel * G + loff
                    val = plsc.load_gather(kwtab, [gidx])
                    rows_v[pl.ds(u * L * G + pp * L, L)] = val
            return carry2

        lax.fori_loop(0, _KWCH // L // 4, u_step, 0)
        pltpu.sync_copy(rows_v, out_hbm.at[pl.ds(e0 * G, _KWCH * G)])
        return carry

    lax.fori_loop(0, per_w // _KWCH, chunk, 0)


def _gather_kw_stage(kw_flat, idx_flat):
    mesh = plsc.VectorSubcoreMesh(core_axis_name="c", subcore_axis_name="s")
    return pl.kernel(
        _gather_kw_body,
        out_type=jax.ShapeDtypeStruct((E * G,), jnp.float32),
        mesh=mesh,
        scratch_types=[
            pltpu.VMEM((N * G,), jnp.float32),
            pltpu.VMEM((_KWCH,), jnp.int32),
            pltpu.VMEM((_KWCH * G,), jnp.float32),
            pltpu.SemaphoreType.DMA,
        ],
        compiler_params=pltpu.CompilerParams(needs_layout_passes=False),
    )(kw_flat, idx_flat)


_BN2 = 1000
_NB2 = N // _BN2


def _lane_rot(x, sh):
    shp = x.shape
    n = shp[-1]
    sq = tuple(d for d in shp[:-1] if d != 1) + (n,)
    xs = jnp.reshape(x, sq)
    iota = lax.broadcasted_iota(jnp.int32, (1,) * (len(sq) - 1) + (n,),
                                len(sq) - 1)
    idx = (iota + sh) % n
    r = jnp.take_along_axis(xs, jnp.broadcast_to(idx, sq),
                            axis=len(sq) - 1, mode="promise_in_bounds")
    return jnp.reshape(r, shp)


def _fold16(x, op):
    for sh in (8, 16, 32, 64):
        x = op(x, _lane_rot(x, sh))
    return x


def _stats_body(gkw_ref, qf_ref, gbe_ref, out_ref, acc_ref):
    i = pl.program_id(0)

    @pl.when(i == 0)
    def _():
        acc_ref[...] = jnp.zeros_like(acc_ref)

    a = gkw_ref[...] - qf_ref[...][:, None, :]
    ps = jnp.sum(jnp.sum(a, axis=0), axis=0, keepdims=True)
    ps2 = jnp.sum(jnp.sum(a * a, axis=0), axis=0, keepdims=True)
    acc_ref[0:1, :] += ps
    acc_ref[1:2, :] += ps2

    @pl.when(i == _NB2 - 1)
    def _():
        accf = _fold16(acc_ref[...], jnp.add)
        mean = accf[0:1, :] / E
        var = accf[1:2, :] / E - mean * mean
        ws = gbe_ref[0:1, :] * lax.rsqrt(var + EPS)
        wsh = gbe_ref[1:2, :] - ws * mean
        out_ref[...] = jnp.concatenate(
            [ws, wsh, jnp.zeros_like(ws), jnp.zeros_like(ws)] * 2, axis=0)


def _stats_stage(gkw2, qft, gbe):
    return pl.pallas_call(
        _stats_body,
        grid=(_NB2,),
        in_specs=[
            pl.BlockSpec((_BN2, 2, C), lambda i: (i, 0, 0)),
            pl.BlockSpec((_BN2, C), lambda i: (i, 0)),
            pl.BlockSpec((2, C), lambda i: (0, 0)),
        ],
        out_specs=pl.BlockSpec((8, C), lambda i: (0, 0)),
        out_shape=jax.ShapeDtypeStruct((8, C), jnp.float32),
        scratch_shapes=[pltpu.VMEM((8, C), jnp.float32)],
    )(gkw2, qft, gbe)


def _weights_body(gkw_ref, qf_ref, st_ref, m2_ref, sm_ref, b2_ref, out_ref):
    wscale = st_ref[0:1, :][:, None, :]
    wshift = st_ref[1:2, :][:, None, :]
    a = gkw_ref[...] - qf_ref[...][:, None, :]
    h = jax.nn.relu(a * wscale + wshift)
    h2 = jnp.reshape(h, (_BN2 * 2, C))
    logits2 = lax.dot_general(h2, m2_ref[...], (((1,), (0,)), ((), ())),
                              preferred_element_type=jnp.float32) + b2_ref[...]
    logits = jnp.reshape(logits2, (_BN2, 2, C))
    ex = jnp.exp(logits)
    es = ex[:, 0, :] + ex[:, 1, :]
    s = lax.dot_general(es, sm_ref[...], (((1,), (0,)), ((), ())),
                        preferred_element_type=jnp.float32)
    out_ref[...] = ex / s[:, None, :]


def _weights_stage(gkw2, qft, stats, m2, sm, b2t):
    return pl.pallas_call(
        _weights_body,
        grid=(_NB2,),
        in_specs=[
            pl.BlockSpec((_BN2, 2, C), lambda i: (i, 0, 0)),
            pl.BlockSpec((_BN2, C), lambda i: (i, 0)),
            pl.BlockSpec((8, C), lambda i: (0, 0)),
            pl.BlockSpec((C, C), lambda i: (0, 0)),
            pl.BlockSpec((C, C), lambda i: (0, 0)),
            pl.BlockSpec((1, C), lambda i: (0, 0)),
        ],
        out_specs=pl.BlockSpec((_BN2, 2, C), lambda i: (i, 0, 0)),
        out_shape=jax.ShapeDtypeStruct((N, 2, C), jnp.float32),
    )(gkw2, qft, stats, m2, sm, b2t)


_AGG_ROWS = 2
_AGG_EDGES = _AGG_ROWS * 128
_AGG_NODES = _AGG_EDGES // K
_AGG_NCHUNK = N // _AGG_NODES


def _agg_body(v_hbm, idx_hbm, w_hbm, out_hbm,
              idx_a, idx_b, vbuf_a, vbuf_b, wbuf_a, wbuf_b, obuf,
              sem_a, sem_b):
    info = _sc_info()
    nc = info.num_cores
    nw = nc * info.num_subcores
    wid = lax.axis_index("s") * nc + lax.axis_index("c")
    bconsts = [jnp.full((L,), j, jnp.int32) for j in range(2 * G)]

    def prefetch(c, idx_v, vbuf, wbuf, sem):
        e0 = c * _AGG_EDGES
        pltpu.sync_copy(idx_hbm.at[pl.ds(e0, _AGG_EDGES)], idx_v)
        for j in range(_AGG_ROWS):
            pltpu.async_copy(v_hbm.at[idx_v.at[pl.ds(j * 128, 128)]],
                             vbuf.at[pl.ds(j * 128, 128)], sem)
        pltpu.async_copy(w_hbm.at[pl.ds(e0 * G, _AGG_EDGES * G)], wbuf, sem)

    def wait_prefetch(idx_v, vbuf, wbuf, sem):
        for j in range(_AGG_ROWS):
            pltpu.make_async_copy(v_hbm.at[idx_v.at[pl.ds(j * 128, 128)]],
                                  vbuf.at[pl.ds(j * 128, 128)], sem).wait()
        pltpu.make_async_copy(w_hbm.at[pl.ds(0, _AGG_EDGES * G)], wbuf,
                              sem).wait()

    def compute(c, vbuf, wbuf):
        def node(ni, carry2):
            accs = [jnp.zeros((L,), jnp.float32) for _ in range(G)]
            for t2 in range(K // 2):
                r = ni * K + 2 * t2
                wv = wbuf[pl.ds(r * G, L)]
                for g in range(G):
                    b0 = jnp.take_along_axis(wv, bconsts[g], axis=0,
                                             mode="promise_in_bounds")
                    b1 = jnp.take_along_axis(wv, bconsts[G + g], axis=0,
                                             mode="promise_in_bounds")
                    accs[g] = (accs[g]
                               + b0 * vbuf[r, pl.ds(g * L, L)]
                               + b1 * vbuf[r + 1, pl.ds(g * L, L)])
            for g in range(G):
                obuf[pl.ds(ni * C + g * L, L)] = accs[g]
            return carry2

        lax.fori_loop(0, _AGG_NODES, node, 0)
        pltpu.sync_copy(obuf, out_hbm.at[pl.ds(c * _AGG_NODES * C,
                                               _AGG_NODES * C)])

    @pl.when(wid < _AGG_NCHUNK)
    def _():
        prefetch(wid, idx_a, vbuf_a, wbuf_a, sem_a)

    pairs = (_AGG_NCHUNK + 2 * nw - 1) // (2 * nw)

    def pair(tt, carry):
        c_a = wid + (2 * tt) * nw
        c_b = c_a + nw
        c_n = c_a + 2 * nw

        @pl.when(c_b < _AGG_NCHUNK)
        def _():
            prefetch(c_b, idx_b, vbuf_b, wbuf_b, sem_b)

        @pl.when(c_a < _AGG_NCHUNK)
        def _():
            wait_prefetch(idx_a, vbuf_a, wbuf_a, sem_a)
            compute(c_a, vbuf_a, wbuf_a)

        @pl.when(c_n < _AGG_NCHUNK)
        def _():
            prefetch(c_n, idx_a, vbuf_a, wbuf_a, sem_a)

        @pl.when(c_b < _AGG_NCHUNK)
        def _():
            wait_prefetch(idx_b, vbuf_b, wbuf_b, sem_b)
            compute(c_b, vbuf_b, wbuf_b)

        return carry

    lax.fori_loop(0, pairs, pair, 0)


def _agg_stage(v, idx_flat, wflat):
    mesh = plsc.VectorSubcoreMesh(core_axis_name="c", subcore_axis_name="s")
    return pl.kernel(
        _agg_body,
        out_type=jax.ShapeDtypeStruct((N * C,), jnp.float32),
        mesh=mesh,
        scratch_types=[
            pltpu.VMEM((_AGG_EDGES,), jnp.int32),
            pltpu.VMEM((_AGG_EDGES,), jnp.int32),
            pltpu.VMEM((_AGG_EDGES, C), jnp.float32),
            pltpu.VMEM((_AGG_EDGES, C), jnp.float32),
            pltpu.VMEM((_AGG_EDGES * G,), jnp.float32),
            pltpu.VMEM((_AGG_EDGES * G,), jnp.float32),
            pltpu.VMEM((_AGG_NODES * C,), jnp.float32),
            pltpu.SemaphoreType.DMA,
            pltpu.SemaphoreType.DMA,
        ],
    )(v, idx_flat, wflat)


def kernel(feat, coord, knn_indexes, params):
    del coord
    p = params
    idx_flat = jnp.reshape(knn_indexes.astype(jnp.int32), (E,))

    v, kw8, qft = _dense_stage(feat, p)
    gkw = _gather_kw_stage(jnp.reshape(kw8, (-1,)), idx_flat)
    gkw2 = jnp.reshape(gkw, (N, 2, C))

    gbe = jnp.stack([jnp.tile(p["g_w"], L), jnp.tile(p["be_w"], L)])
    m2 = jnp.kron(jnp.eye(L, dtype=jnp.float32), p["W_w2"])
    sm = jnp.kron(jnp.ones((L, L), jnp.float32), jnp.eye(G, dtype=jnp.float32))
    b2t = jnp.tile(p["b_w2"], L).reshape(1, C)

    stats = _stats_stage(gkw2, qft, gbe)
    w = _weights_stage(gkw2, qft, stats, m2, sm, b2t)

    out = _agg_stage(v, idx_flat, jnp.reshape(w, (-1,)))
    return jnp.reshape(out, (N, C))


# speedup vs baseline: 10.8487x; 1.0000x over previous
import functools

import jax
import jax.numpy as jnp
from jax import lax
from jax.experimental import pallas as pl
from jax.experimental.pallas import tpu as pltpu
from jax.experimental.pallas import tpu_sc as plsc

N = 10000
K = 32
C = 128
G = 8
L = 16
EPS = 1e-5
E = N * K

_INFO = None


def _sc_info():
    global _INFO
    if _INFO is None:
        _INFO = plsc.get_sparse_core_info()
    return _INFO


def _dense_body(feat_ref, wq_ref, wk_ref, wv_ref, cq_ref, ck_ref, cv_ref,
                w1_ref, w1t_ref, b1_ref,
                v_ref, kw_ref, qf_ref):
    f = feat_ref[...]
    colmean = jnp.sum(f, axis=0, keepdims=True) / N
    xtx = lax.dot_general(f, f, (((0,), (0,)), ((), ())),
                          preferred_element_type=jnp.float32) / N

    def proj_bn_relu(W, g, be):
        mu0 = colmean @ W
        var = jnp.sum((xtx @ W) * W, axis=0, keepdims=True) - mu0 * mu0
        sc = g * lax.rsqrt(var + EPS)
        y = lax.dot_general(f, W * sc, (((1,), (0,)), ((), ())),
                            preferred_element_type=jnp.float32)
        return jax.nn.relu(y - mu0 * sc + be)

    cq = cq_ref[...]
    ck = ck_ref[...]
    q = proj_bn_relu(wq_ref[...], cq[1:2], cq[2:3])
    k = proj_bn_relu(wk_ref[...], ck[1:2], ck[2:3])
    cv = cv_ref[...]
    v = lax.dot_general(f, wv_ref[...], (((1,), (0,)), ((), ())),
                        preferred_element_type=jnp.float32) + cv[0:1]
    v_ref[...] = v
    w1 = w1_ref[...]
    kw_ref[...] = lax.dot_general(k, w1, (((1,), (0,)), ((), ())),
                                  preferred_element_type=jnp.float32)
    qf_ref[...] = lax.dot_general(q, w1t_ref[...], (((1,), (0,)), ((), ())),
                                  preferred_element_type=jnp.float32) - b1_ref[...]


def _dense_stage(feat, p):
    cq = jnp.stack([p["b_q"], p["g_q"], p["be_q"], jnp.zeros((C,), jnp.float32)])
    ck = jnp.stack([p["b_k"], p["g_k"], p["be_k"], jnp.zeros((C,), jnp.float32)])
    cv = jnp.stack([p["b_v"], jnp.zeros((C,), jnp.float32),
                    jnp.zeros((C,), jnp.float32), jnp.zeros((C,), jnp.float32)])
    return pl.pallas_call(
        _dense_body,
        out_shape=(
            jax.ShapeDtypeStruct((N, C), jnp.float32),
            jax.ShapeDtypeStruct((N, G), jnp.float32),
            jax.ShapeDtypeStruct((N, C), jnp.float32),
        ),
    )(feat, p["W_q"], p["W_k"], p["W_v"], cq, ck, cv,
      p["W_w1"], jnp.tile(p["W_w1"], (1, L)),
      jnp.tile(p["b_w1"], L).reshape(1, C))


_KWCH = 2000


def _gather_kw_body(kw_hbm, idx_hbm, out_hbm, kwtab, idx_v, rows_v, sem):
    info = _sc_info()
    nc = info.num_cores
    nw = nc * info.num_subcores
    wid = lax.axis_index("s") * nc + lax.axis_index("c")
    per_w = E // nw
    base = wid * per_w
    pltpu.sync_copy(kw_hbm, kwtab)
    loff = jax.lax.broadcasted_iota(jnp.int32, (L,), 0) % G
    halfsel = jax.lax.broadcasted_iota(jnp.int32, (L,), 0) // G
    pconsts = [halfsel + (2 * pp) for pp in range(G)]

    def chunk(cc, carry):
        e0 = base + cc * _KWCH
        pltpu.sync_copy(idx_hbm.at[pl.ds(e0, _KWCH)], idx_v)

        def u_step(u4, carry2):
            for uu in range(5):
                u = u4 * 5 + uu
                idx16 = idx_v[pl.ds(u * L, L)]
                for pp in range(G):
                    sel = jnp.take_along_axis(idx16, pconsts[pp], axis=0,
                                              mode="promise_in_bounds")
                    gidx = sel * G + loff
                    val = plsc.load_gather(kwtab, [gidx])
                    rows_v[pl.ds(u * L * G + pp * L, L)] = val
            return carry2

        lax.fori_loop(0, _KWCH // L // 5, u_step, 0)
        pltpu.sync_copy(rows_v, out_hbm.at[pl.ds(e0 * G, _KWCH * G)])
        return carry

    lax.fori_loop(0, per_w // _KWCH, chunk, 0)


def _gather_kw_stage(kw_flat, idx_flat):
    mesh = plsc.VectorSubcoreMesh(core_axis_name="c", subcore_axis_name="s")
    return pl.kernel(
        _gather_kw_body,
        out_type=jax.ShapeDtypeStruct((E * G,), jnp.float32),
        mesh=mesh,
        scratch_types=[
            pltpu.VMEM((N * G,), jnp.float32),
            pltpu.VMEM((_KWCH,), jnp.int32),
            pltpu.VMEM((_KWCH * G,), jnp.float32),
            pltpu.SemaphoreType.DMA,
        ],
        compiler_params=pltpu.CompilerParams(needs_layout_passes=False),
    )(kw_flat, idx_flat)


_BN2 = 1000
_NB2 = N // _BN2


def _lane_rot(x, sh):
    shp = x.shape
    n = shp[-1]
    sq = tuple(d for d in shp[:-1] if d != 1) + (n,)
    xs = jnp.reshape(x, sq)
    iota = lax.broadcasted_iota(jnp.int32, (1,) * (len(sq) - 1) + (n,),
                                len(sq) - 1)
    idx = (iota + sh) % n
    r = jnp.take_along_axis(xs, jnp.broadcast_to(idx, sq),
                            axis=len(sq) - 1, mode="promise_in_bounds")
    return jnp.reshape(r, shp)


def _fold16(x, op):
    for sh in (8, 16, 32, 64):
        x = op(x, _lane_rot(x, sh))
    return x


def _stats_body(gkw_ref, qf_ref, gbe_ref, out_ref, acc_ref):
    i = pl.program_id(0)

    @pl.when(i == 0)
    def _():
        acc_ref[...] = jnp.zeros_like(acc_ref)

    a = gkw_ref[...] - qf_ref[...][:, None, :]
    ps = jnp.sum(jnp.sum(a, axis=0), axis=0, keepdims=True)
    ps2 = jnp.sum(jnp.sum(a * a, axis=0), axis=0, keepdims=True)
    acc_ref[0:1, :] += ps
    acc_ref[1:2, :] += ps2

    @pl.when(i == _NB2 - 1)
    def _():
        accf = _fold16(acc_ref[...], jnp.add)
        mean = accf[0:1, :] / E
        var = accf[1:2, :] / E - mean * mean
        ws = gbe_ref[0:1, :] * lax.rsqrt(var + EPS)
        wsh = gbe_ref[1:2, :] - ws * mean
        out_ref[...] = jnp.concatenate(
            [ws, wsh, jnp.zeros_like(ws), jnp.zeros_like(ws)] * 2, axis=0)


def _stats_stage(gkw2, qft, gbe):
    return pl.pallas_call(
        _stats_body,
        grid=(_NB2,),
        in_specs=[
            pl.BlockSpec((_BN2, 2, C), lambda i: (i, 0, 0)),
            pl.BlockSpec((_BN2, C), lambda i: (i, 0)),
            pl.BlockSpec((2, C), lambda i: (0, 0)),
        ],
        out_specs=pl.BlockSpec((8, C), lambda i: (0, 0)),
        out_shape=jax.ShapeDtypeStruct((8, C), jnp.float32),
        scratch_shapes=[pltpu.VMEM((8, C), jnp.float32)],
    )(gkw2, qft, gbe)


def _weights_body(gkw_ref, qf_ref, st_ref, m2_ref, sm_ref, b2_ref, out_ref):
    wscale = st_ref[0:1, :][:, None, :]
    wshift = st_ref[1:2, :][:, None, :]
    a = gkw_ref[...] - qf_ref[...][:, None, :]
    h = jax.nn.relu(a * wscale + wshift)
    h2 = jnp.reshape(h, (_BN2 * 2, C))
    logits2 = lax.dot_general(h2, m2_ref[...], (((1,), (0,)), ((), ())),
                              preferred_element_type=jnp.float32) + b2_ref[...]
    logits = jnp.reshape(logits2, (_BN2, 2, C))
    ex = jnp.exp(logits)
    es = ex[:, 0, :] + ex[:, 1, :]
    s = lax.dot_general(es, sm_ref[...], (((1,), (0,)), ((), ())),
                        preferred_element_type=jnp.float32)
    out_ref[...] = ex / s[:, None, :]


def _weights_stage(gkw2, qft, stats, m2, sm, b2t):
    return pl.pallas_call(
        _weights_body,
        grid=(_NB2,),
        in_specs=[
            pl.BlockSpec((_BN2, 2, C), lambda i: (i, 0, 0)),
            pl.BlockSpec((_BN2, C), lambda i: (i, 0)),
            pl.BlockSpec((8, C), lambda i: (0, 0)),
            pl.BlockSpec((C, C), lambda i: (0, 0)),
            pl.BlockSpec((C, C), lambda i: (0, 0)),
            pl.BlockSpec((1, C), lambda i: (0, 0)),
        ],
        out_specs=pl.BlockSpec((_BN2, 2, C), lambda i: (i, 0, 0)),
        out_shape=jax.ShapeDtypeStruct((N, 2, C), jnp.float32),
    )(gkw2, qft, stats, m2, sm, b2t)


_AGG_ROWS = 2
_AGG_EDGES = _AGG_ROWS * 128
_AGG_NODES = _AGG_EDGES // K
_AGG_NCHUNK = N // _AGG_NODES


def _agg_body(v_hbm, idx_hbm, w_hbm, out_hbm,
              idx_a, idx_b, vbuf_a, vbuf_b, wbuf_a, wbuf_b, obuf,
              sem_a, sem_b):
    info = _sc_info()
    nc = info.num_cores
    nw = nc * info.num_subcores
    wid = lax.axis_index("s") * nc + lax.axis_index("c")
    bconsts = [jnp.full((L,), j, jnp.int32) for j in range(2 * G)]

    def prefetch(c, idx_v, vbuf, wbuf, sem):
        e0 = c * _AGG_EDGES
        pltpu.sync_copy(idx_hbm.at[pl.ds(e0, _AGG_EDGES)], idx_v)
        for j in range(_AGG_ROWS):
            pltpu.async_copy(v_hbm.at[idx_v.at[pl.ds(j * 128, 128)]],
                             vbuf.at[pl.ds(j * 128, 128)], sem)
        pltpu.async_copy(w_hbm.at[pl.ds(e0 * G, _AGG_EDGES * G)], wbuf, sem)

    def wait_prefetch(idx_v, vbuf, wbuf, sem):
        for j in range(_AGG_ROWS):
            pltpu.make_async_copy(v_hbm.at[idx_v.at[pl.ds(j * 128, 128)]],
                                  vbuf.at[pl.ds(j * 128, 128)], sem).wait()
        pltpu.make_async_copy(w_hbm.at[pl.ds(0, _AGG_EDGES * G)], wbuf,
                              sem).wait()

    def compute(c, vbuf, wbuf):
        def node(ni, carry2):
            accs = [jnp.zeros((L,), jnp.float32) for _ in range(G)]
            for t2 in range(K // 2):
                r = ni * K + 2 * t2
                wv = wbuf[pl.ds(r * G, L)]
                for g in range(G):
                    b0 = jnp.take_along_axis(wv, bconsts[g], axis=0,
                                             mode="promise_in_bounds")
                    b1 = jnp.take_along_axis(wv, bconsts[G + g], axis=0,
                                             mode="promise_in_bounds")
                    accs[g] = (accs[g]
                               + b0 * vbuf[r, pl.ds(g * L, L)]
                               + b1 * vbuf[r + 1, pl.ds(g * L, L)])
            for g in range(G):
                obuf[pl.ds(ni * C + g * L, L)] = accs[g]
            return carry2

        lax.fori_loop(0, _AGG_NODES, node, 0)
        pltpu.sync_copy(obuf, out_hbm.at[pl.ds(c * _AGG_NODES * C,
                                               _AGG_NODES * C)])

    @pl.when(wid < _AGG_NCHUNK)
    def _():
        prefetch(wid, idx_a, vbuf_a, wbuf_a, sem_a)

    pairs = (_AGG_NCHUNK + 2 * nw - 1) // (2 * nw)

    def pair(tt, carry):
        c_a = wid + (2 * tt) * nw
        c_b = c_a + nw
        c_n = c_a + 2 * nw

        @pl.when(c_b < _AGG_NCHUNK)
        def _():
            prefetch(c_b, idx_b, vbuf_b, wbuf_b, sem_b)

        @pl.when(c_a < _AGG_NCHUNK)
        def _():
            wait_prefetch(idx_a, vbuf_a, wbuf_a, sem_a)
            compute(c_a, vbuf_a, wbuf_a)

        @pl.when(c_n < _AGG_NCHUNK)
        def _():
            prefetch(c_n, idx_a, vbuf_a, wbuf_a, sem_a)

        @pl.when(c_b < _AGG_NCHUNK)
        def _():
            wait_prefetch(idx_b, vbuf_b, wbuf_b, sem_b)
            compute(c_b, vbuf_b, wbuf_b)

        return carry

    lax.fori_loop(0, pairs, pair, 0)


def _agg_stage(v, idx_flat, wflat):
    mesh = plsc.VectorSubcoreMesh(core_axis_name="c", subcore_axis_name="s")
    return pl.kernel(
        _agg_body,
        out_type=jax.ShapeDtypeStruct((N * C,), jnp.float32),
        mesh=mesh,
        scratch_types=[
            pltpu.VMEM((_AGG_EDGES,), jnp.int32),
            pltpu.VMEM((_AGG_EDGES,), jnp.int32),
            pltpu.VMEM((_AGG_EDGES, C), jnp.float32),
            pltpu.VMEM((_AGG_EDGES, C), jnp.float32),
            pltpu.VMEM((_AGG_EDGES * G,), jnp.float32),
            pltpu.VMEM((_AGG_EDGES * G,), jnp.float32),
            pltpu.VMEM((_AGG_NODES * C,), jnp.float32),
            pltpu.SemaphoreType.DMA,
            pltpu.SemaphoreType.DMA,
        ],
    )(v, idx_flat, wflat)


def kernel(feat, coord, knn_indexes, params):
    del coord
    p = params
    idx_flat = jnp.reshape(knn_indexes.astype(jnp.int32), (E,))

    v, kw8, qft = _dense_stage(feat, p)
    gkw = _gather_kw_stage(jnp.reshape(kw8, (-1,)), idx_flat)
    gkw2 = jnp.reshape(gkw, (N, 2, C))

    gbe = jnp.stack([jnp.tile(p["g_w"], L), jnp.tile(p["be_w"], L)])
    m2 = jnp.kron(jnp.eye(L, dtype=jnp.float32), p["W_w2"])
    sm = jnp.kron(jnp.ones((L, L), jnp.float32), jnp.eye(G, dtype=jnp.float32))
    b2t = jnp.tile(p["b_w2"], L).reshape(1, C)

    stats = _stats_stage(gkw2, qft, gbe)
    w = _weights_stage(gkw2, qft, stats, m2, sm, b2t)

    out = _agg_stage(v, idx_flat, jnp.reshape(w, (-1,)))
    return jnp.reshape(out, (N, C))
